# baseline (speedup 1.0000x reference)
import functools

import jax
import jax.numpy as jnp
from jax import lax
from jax.experimental import pallas as pl
from jax.experimental.pallas import tpu as pltpu
from jax.experimental.pallas import tpu_sc as plsc

R, S, D = 16384, 50, 64
DP = 128
B = R * S
NC, NS = 2, 16
NW = NC * NS
B_PER_W = B // NW
CHUNK = 200
SET = 2
NBUF = 2 * SET
N_CHUNKS = B_PER_W // CHUNK
N_GROUPS = N_CHUNKS // SET

_mesh = plsc.VectorSubcoreMesh(core_axis_name="c", subcore_axis_name="s")


@functools.partial(
    pl.kernel,
    mesh=_mesh,
    out_type=jax.ShapeDtypeStruct((B, DP), jnp.float32),
    scratch_types=(
        [pltpu.VMEM((B_PER_W,), jnp.int32)]
        + [pltpu.VMEM((CHUNK, DP), jnp.float32) for _ in range(NBUF)]
        + [pltpu.SemaphoreType.DMA for _ in range(2 * NBUF)]
    ),
    compiler_params=pltpu.CompilerParams(use_tc_tiling_on_sc=True),
)
def _gather(ids_hbm, table_hbm, out_hbm, idx_v, *rest):
    rows = rest[:NBUF]
    gsem = rest[NBUF:2 * NBUF]
    osem = rest[2 * NBUF:]
    wid = lax.axis_index("s") * NC + lax.axis_index("c")
    base = wid * B_PER_W

    def g_start(b, chunk_i):
        offs = idx_v.at[pl.ds(chunk_i * CHUNK, CHUNK)]
        pltpu.async_copy(table_hbm.at[offs], rows[b], gsem[b])

    def g_wait(b):
        offs = idx_v.at[pl.ds(0, CHUNK)]
        pltpu.make_async_copy(table_hbm.at[offs], rows[b], gsem[b]).wait()

    def o_start(b, chunk_i):
        off = base + chunk_i * CHUNK
        pltpu.async_copy(rows[b], out_hbm.at[pl.ds(off, CHUNK)], osem[b])

    def o_wait(b):
        pltpu.make_async_copy(
            rows[b], out_hbm.at[pl.ds(base, CHUNK)], osem[b]).wait()

    pltpu.sync_copy(ids_hbm.at[pl.ds(base, B_PER_W)], idx_v)
    for i in range(SET):
        g_start(i, i)

    def body(g, carry):
        def recycle(tset, gather_base):
            for i in range(SET):
                b = tset * SET + i

                @pl.when(g > 0)
                def _():
                    o_wait(b)

                @pl.when(g + 1 < N_GROUPS)
                def _():
                    g_start(b, gather_base + i)

        def flush(sset, out_base):
            for i in range(SET):
                b = sset * SET + i
                g_wait(b)
                o_start(b, out_base + i)

        @pl.when(g % 2 == 0)
        def _():
            recycle(1, (g + 1) * SET)
            flush(0, g * SET)

        @pl.when(g % 2 == 1)
        def _():
            recycle(0, (g + 1) * SET)
            flush(1, g * SET)

        return carry

    lax.fori_loop(0, N_GROUPS, body, 0)
    last_set = (N_GROUPS - 1) % 2
    for i in range(SET):
        o_wait(last_set * SET + i)


def kernel(ids, emb_var):
    table = jnp.pad(emb_var, ((0, 0), (0, DP - D)))
    out = _gather(ids.reshape(-1), table)
    return out[:, :D].reshape(R, S, D)


# speedup vs baseline: 1.3946x; 1.3946x over previous
import functools

import jax
import jax.numpy as jnp
from jax import lax
from jax.experimental import pallas as pl
from jax.experimental.pallas import tpu as pltpu
from jax.experimental.pallas import tpu_sc as plsc

R, S, D = 16384, 50, 64
DP = 128
SP = 64
NC, NS = 2, 16
NW = NC * NS
R_PER_W = R // NW
SET = 2
NBUF = 2 * SET
N_GROUPS = R_PER_W // SET

_mesh = plsc.VectorSubcoreMesh(core_axis_name="c", subcore_axis_name="s")


@functools.partial(
    pl.kernel,
    mesh=_mesh,
    out_type=jax.ShapeDtypeStruct((R, S, DP), jnp.float32),
    scratch_types=(
        [pltpu.VMEM((R_PER_W * SP,), jnp.int32)]
        + [pltpu.VMEM((S, DP), jnp.float32) for _ in range(NBUF)]
        + [pltpu.SemaphoreType.DMA for _ in range(2 * NBUF)]
    ),
    compiler_params=pltpu.CompilerParams(use_tc_tiling_on_sc=True),
)
def _gather(ids_hbm, table_hbm, out_hbm, idx_v, *rest):
    rows = rest[:NBUF]
    gsem = rest[NBUF:2 * NBUF]
    osem = rest[2 * NBUF:]
    wid = lax.axis_index("s") * NC + lax.axis_index("c")
    base = wid * R_PER_W

    def g_start(b, row_i):
        offs = idx_v.at[pl.ds(row_i * SP, S)]
        pltpu.async_copy(table_hbm.at[offs], rows[b], gsem[b])

    def g_wait(b):
        offs = idx_v.at[pl.ds(0, S)]
        pltpu.make_async_copy(table_hbm.at[offs], rows[b], gsem[b]).wait()

    def o_start(b, row_i):
        pltpu.async_copy(rows[b], out_hbm.at[base + row_i], osem[b])

    def o_wait(b):
        pltpu.make_async_copy(rows[b], out_hbm.at[base], osem[b]).wait()

    pltpu.sync_copy(ids_hbm.at[pl.ds(base * SP, R_PER_W * SP)], idx_v)
    for i in range(SET):
        g_start(i, i)

    def body(g, carry):
        def recycle(tset, gather_base):
            for i in range(SET):
                b = tset * SET + i

                @pl.when(g > 0)
                def _():
                    o_wait(b)

                @pl.when(g + 1 < N_GROUPS)
                def _():
                    g_start(b, gather_base + i)

        def flush(sset, out_base):
            for i in range(SET):
                b = sset * SET + i
                g_wait(b)
                o_start(b, out_base + i)

        @pl.when(g % 2 == 0)
        def _():
            recycle(1, (g + 1) * SET)
            flush(0, g * SET)

        @pl.when(g % 2 == 1)
        def _():
            recycle(0, (g + 1) * SET)
            flush(1, g * SET)

        return carry

    lax.fori_loop(0, N_GROUPS, body, 0)
    last_set = (N_GROUPS - 1) % 2
    for i in range(SET):
        o_wait(last_set * SET + i)


def kernel(ids, emb_var):
    table = jnp.pad(emb_var, ((0, 0), (0, DP - D)))
    flat_ids = jnp.pad(ids, ((0, 0), (0, SP - S))).reshape(-1)
    out = _gather(flat_ids, table)
    return out[:, :, :D]


# speedup vs baseline: 1.4012x; 1.0048x over previous
import functools

import jax
import jax.numpy as jnp
from jax import lax
from jax.experimental import pallas as pl
from jax.experimental.pallas import tpu as pltpu
from jax.experimental.pallas import tpu_sc as plsc

R, S, D = 16384, 50, 64
DP = 128
SP = 64
NC, NS = 2, 16
NW = NC * NS
R_PER_W = R // NW
SET = 4
NBUF = 2 * SET
N_GROUPS = R_PER_W // SET

_mesh = plsc.VectorSubcoreMesh(core_axis_name="c", subcore_axis_name="s")


@functools.partial(
    pl.kernel,
    mesh=_mesh,
    out_type=jax.ShapeDtypeStruct((R, S, DP), jnp.float32),
    scratch_types=(
        [pltpu.VMEM((R_PER_W * SP,), jnp.int32)]
        + [pltpu.VMEM((S, DP), jnp.float32) for _ in range(NBUF)]
        + [pltpu.SemaphoreType.DMA for _ in range(2 * NBUF)]
    ),
    compiler_params=pltpu.CompilerParams(use_tc_tiling_on_sc=True),
)
def _gather(ids_hbm, table_hbm, out_hbm, idx_v, *rest):
    rows = rest[:NBUF]
    gsem = rest[NBUF:2 * NBUF]
    osem = rest[2 * NBUF:]
    wid = lax.axis_index("s") * NC + lax.axis_index("c")
    base = wid * R_PER_W

    def g_start(b, row_i):
        offs = idx_v.at[pl.ds(row_i * SP, S)]
        pltpu.async_copy(table_hbm.at[offs], rows[b], gsem[b])

    def g_wait(b):
        offs = idx_v.at[pl.ds(0, S)]
        pltpu.make_async_copy(table_hbm.at[offs], rows[b], gsem[b]).wait()

    def o_start(b, row_i):
        pltpu.async_copy(rows[b], out_hbm.at[base + row_i], osem[b])

    def o_wait(b):
        pltpu.make_async_copy(rows[b], out_hbm.at[base], osem[b]).wait()

    pltpu.sync_copy(ids_hbm.at[pl.ds(base * SP, R_PER_W * SP)], idx_v)
    for i in range(SET):
        g_start(i, i)

    def body(g, carry):
        def recycle(tset, gather_base):
            for i in range(SET):
                b = tset * SET + i

                @pl.when(g > 0)
                def _():
                    o_wait(b)

                @pl.when(g + 1 < N_GROUPS)
                def _():
                    g_start(b, gather_base + i)

        def flush(sset, out_base):
            for i in range(SET):
                b = sset * SET + i
                g_wait(b)
                o_start(b, out_base + i)

        @pl.when(g % 2 == 0)
        def _():
            recycle(1, (g + 1) * SET)
            flush(0, g * SET)

        @pl.when(g % 2 == 1)
        def _():
            recycle(0, (g + 1) * SET)
            flush(1, g * SET)

        return carry

    lax.fori_loop(0, N_GROUPS, body, 0)
    last_set = (N_GROUPS - 1) % 2
    for i in range(SET):
        o_wait(last_set * SET + i)


def kernel(ids, emb_var):
    table = jnp.pad(emb_var, ((0, 0), (0, DP - D)))
    flat_ids = jnp.pad(ids, ((0, 0), (0, SP - S))).reshape(-1)
    out = _gather(flat_ids, table)
    return out[:, :, :D]
